# baseline (speedup 1.0000x reference)
import functools

import jax
import jax.numpy as jnp
from jax import lax
from jax.experimental import pallas as pl
from jax.experimental.pallas import tpu as pltpu
from jax.experimental.pallas import tpu_sc as plsc

HIDDEN = 768
EPS = 1e-12
NC, NS = 2, 16
NW = NC * NS
CHUNK = 64
TCBLK = 256


@functools.lru_cache(maxsize=None)
def _build_gather(ntok):
    tok_per_w = ntok // NW
    nchunk = tok_per_w // CHUNK
    mesh = plsc.VectorSubcoreMesh(core_axis_name="c", subcore_axis_name="s")

    @functools.partial(
        pl.kernel,
        mesh=mesh,
        out_type=jax.ShapeDtypeStruct((ntok, HIDDEN), jnp.float32),
        scratch_types=[
            pltpu.VMEM((nchunk, CHUNK), jnp.int32),
            pltpu.VMEM((2, CHUNK, HIDDEN), jnp.float32),
            pltpu.SemaphoreType.DMA,
            pltpu.SemaphoreType.DMA,
        ],
    )
    def body(ids_hbm, tab_hbm, out_hbm, idx_v, buf_v, gsem, wsem):
        wid = lax.axis_index("s") * NC + lax.axis_index("c")
        tok0 = wid * tok_per_w
        pltpu.sync_copy(ids_hbm.at[wid], idx_v)

        cur_g = pltpu.async_copy(tab_hbm.at[idx_v.at[0]], buf_v.at[0], gsem)
        prev_w = None
        for g in range(nchunk):
            cur_g.wait()
            if prev_w is not None:
                prev_w.wait()
            prev_w = pltpu.async_copy(
                buf_v.at[g % 2], out_hbm.at[pl.ds(tok0 + g * CHUNK, CHUNK)],
                wsem)
            if g + 1 < nchunk:
                cur_g = pltpu.async_copy(
                    tab_hbm.at[idx_v.at[g + 1]], buf_v.at[(g + 1) % 2], gsem)
        prev_w.wait()

    return body


def _ln_body(rows_ref, pos_ref, gam_ref, bet_ref, out_ref):
    b, t, h = rows_ref.shape
    x = (rows_ref[...] + pos_ref[...][None]).reshape(b * t, h)
    ones = jnp.ones((h, 128), jnp.float32)
    s = lax.dot(x, ones, precision=lax.Precision.HIGHEST)[:, 0:1]
    q = lax.dot(x * x, ones, precision=lax.Precision.HIGHEST)[:, 0:1]
    m = s * (1.0 / h)
    var = q * (1.0 / h) - m * m
    inv = lax.rsqrt(var + EPS)
    out = ((x - m) * inv) * gam_ref[...] + bet_ref[...]
    out_ref[...] = out.reshape(b, t, h)


@functools.lru_cache(maxsize=None)
def _build_ln(batch, seq):
    return pl.pallas_call(
        _ln_body,
        grid=(seq // TCBLK,),
        in_specs=[
            pl.BlockSpec((batch, TCBLK, HIDDEN), lambda i: (0, i, 0)),
            pl.BlockSpec((TCBLK, HIDDEN), lambda i: (i, 0)),
            pl.BlockSpec((1, HIDDEN), lambda i: (0, 0)),
            pl.BlockSpec((1, HIDDEN), lambda i: (0, 0)),
        ],
        out_specs=pl.BlockSpec((batch, TCBLK, HIDDEN), lambda i: (0, i, 0)),
        out_shape=jax.ShapeDtypeStruct((batch, seq, HIDDEN), jnp.float32),
    )


def kernel(input_ids, word_embeddings, position_embeddings, ln_gamma, ln_beta):
    b, s = input_ids.shape
    ntok = b * s
    ids = input_ids.astype(jnp.int32).reshape(NW, -1, CHUNK)
    rows = _build_gather(ntok)(ids, word_embeddings)
    return _build_ln(b, s)(rows.reshape(b, s, HIDDEN), position_embeddings,
                           ln_gamma[None], ln_beta[None])


# speedup vs baseline: 1.1242x; 1.1242x over previous
import functools

import jax
import jax.numpy as jnp
from jax import lax
from jax.experimental import pallas as pl
from jax.experimental.pallas import tpu as pltpu
from jax.experimental.pallas import tpu_sc as plsc

HIDDEN = 768
EPS = 1e-12
NC, NS = 2, 16
NW = NC * NS
CHUNK = 64
TCBLK = 256


@functools.lru_cache(maxsize=None)
def _build_gather(ntok):
    tok_per_w = ntok // NW
    nchunk = tok_per_w // CHUNK
    mesh = plsc.VectorSubcoreMesh(core_axis_name="c", subcore_axis_name="s")

    @functools.partial(
        pl.kernel,
        mesh=mesh,
        out_type=jax.ShapeDtypeStruct((ntok, HIDDEN), jnp.float32),
        scratch_types=[
            pltpu.VMEM((nchunk, CHUNK), jnp.int32),
            pltpu.VMEM((2, CHUNK, HIDDEN), jnp.float32),
            pltpu.SemaphoreType.DMA,
            pltpu.SemaphoreType.DMA,
        ],
    )
    def body(ids_hbm, tab_hbm, out_hbm, idx_v, buf_v, gsem, wsem):
        wid = lax.axis_index("s") * NC + lax.axis_index("c")
        tok0 = wid * tok_per_w
        pltpu.sync_copy(ids_hbm.at[wid], idx_v)

        cur_g = pltpu.async_copy(tab_hbm.at[idx_v.at[0]], buf_v.at[0], gsem)
        prev_w = None
        for g in range(nchunk):
            cur_g.wait()
            if prev_w is not None:
                prev_w.wait()
            prev_w = pltpu.async_copy(
                buf_v.at[g % 2], out_hbm.at[pl.ds(tok0 + g * CHUNK, CHUNK)],
                wsem)
            if g + 1 < nchunk:
                cur_g = pltpu.async_copy(
                    tab_hbm.at[idx_v.at[g + 1]], buf_v.at[(g + 1) % 2], gsem)
        prev_w.wait()

    return body


def _ln_body(rows_ref, pos_ref, gam_ref, bet_ref, out_ref):
    x = rows_ref[...] + pos_ref[...]
    ones = jnp.ones((HIDDEN, 128), jnp.float32)
    s = jnp.dot(x, ones)[:, 0:1]
    q = jnp.dot(x * x, ones)[:, 0:1]
    m = s * (1.0 / HIDDEN)
    var = q * (1.0 / HIDDEN) - m * m
    inv = lax.rsqrt(var + EPS)
    out_ref[...] = (x - m) * inv * gam_ref[...] + bet_ref[...]


@functools.lru_cache(maxsize=None)
def _build_ln(ntok, seq):
    nposblk = seq // TCBLK
    return pl.pallas_call(
        _ln_body,
        grid=(ntok // TCBLK,),
        in_specs=[
            pl.BlockSpec((TCBLK, HIDDEN), lambda i: (i, 0)),
            pl.BlockSpec((TCBLK, HIDDEN), lambda i: (i % nposblk, 0)),
            pl.BlockSpec((1, HIDDEN), lambda i: (0, 0)),
            pl.BlockSpec((1, HIDDEN), lambda i: (0, 0)),
        ],
        out_specs=pl.BlockSpec((TCBLK, HIDDEN), lambda i: (i, 0)),
        out_shape=jax.ShapeDtypeStruct((ntok, HIDDEN), jnp.float32),
    )


def kernel(input_ids, word_embeddings, position_embeddings, ln_gamma, ln_beta):
    b, s = input_ids.shape
    ntok = b * s
    ids = input_ids.astype(jnp.int32).reshape(NW, -1, CHUNK)
    rows = _build_gather(ntok)(ids, word_embeddings)
    out = _build_ln(ntok, s)(rows, position_embeddings,
                             ln_gamma[None], ln_beta[None])
    return out.reshape(b, s, HIDDEN)
